# Initial kernel scaffold; baseline (speedup 1.0000x reference)
import jax
import jax.numpy as jnp
from jax.experimental import pallas as pl


def kernel(word_indices, table):
    raise NotImplementedError("write your pallas kernel here")


# speedup vs baseline: 2.4358x; 2.4358x over previous
import functools

import jax
import jax.numpy as jnp
from jax import lax
from jax.experimental import pallas as pl
from jax.experimental.pallas import tpu as pltpu
from jax.experimental.pallas import tpu_sc as plsc

VOCAB = 1_000_000
D = 32
B = 4096
H = 200

NC = 2
NS = 16
NW = NC * NS
BPW = B // NW
IPW = BPW * H
CROWS = 4
CIDX = CROWS * H
NCH = BPW // CROWS
SLICES = [(o, min(128, CIDX - o)) for o in range(0, CIDX, 128)]


def _copies(table_hbm, idx_v, buf, sem, cbase):
    out = []
    for (o, n) in SLICES:
        src = table_hbm.at[idx_v.at[pl.ds(cbase + o, n)]]
        dst = buf.at[pl.ds(o, n)]
        out.append(pltpu.make_async_copy(src, dst, sem))
    return out


def _issue(table_hbm, idx_v, buf, sem, cbase):
    for c in _copies(table_hbm, idx_v, buf, sem, cbase):
        c.start()


def _drain(table_hbm, idx_v, buf, sem, cbase):
    for c in _copies(table_hbm, idx_v, buf, sem, cbase):
        c.wait()


def _accum_row(buf, rbase):
    def body(j, accs):
        a0, a1 = accs
        a0 = a0 + buf[rbase + j, pl.ds(0, 16)]
        a1 = a1 + buf[rbase + j, pl.ds(16, 16)]
        return (a0, a1)

    z = jnp.zeros((16,), jnp.float32)
    return lax.fori_loop(0, H, body, (z, z), unroll=8)


_GATHER_DNUMS = lax.GatherDimensionNumbers(
    offset_dims=(), collapsed_slice_dims=(0,), start_index_map=(0,))


def _perm16(x, perm):
    return lax.gather(x, perm[:, None], _GATHER_DNUMS, (1,),
                      mode=lax.GatherScatterMode.PROMISE_IN_BOUNDS)


def _count_nnz(idx_v, ibase):
    lane = lax.iota(jnp.int32, 16)
    cnt = jnp.zeros((16,), jnp.int32)
    one = jnp.ones((16,), jnp.int32)
    zero = jnp.zeros((16,), jnp.int32)
    for k in range(H // 16 + 1):
        v = idx_v[pl.ds(ibase + 16 * k, 16)]
        ok = v != 0
        if k == H // 16:
            ok = jnp.logical_and(ok, lane < H - 16 * k)
        cnt = cnt + jnp.where(ok, one, zero)
    for s in (1, 2, 4, 8):
        cnt = cnt + _perm16(cnt, lane ^ s)
    return cnt


def _body(idx_hbm, table_hbm, out_hbm, idx_v, buf0, buf1, out_v, t0_v,
          sem0, sem1):
    wid = lax.axis_index("s") * NC + lax.axis_index("c")
    pltpu.sync_copy(idx_hbm.at[wid], idx_v.at[pl.ds(0, IPW)])
    pltpu.sync_copy(table_hbm.at[pl.ds(0, 1)], t0_v)
    t0a = t0_v[0, pl.ds(0, 16)]
    t0b = t0_v[0, pl.ds(16, 16)]
    bufs = (buf0, buf1)
    sems = (sem0, sem1)

    for b in (0, 1):
        _issue(table_hbm, idx_v, bufs[b], sems[b],
               pl.multiple_of(b * CIDX, 8))

    def outer(g, carry):
        for b in (0, 1):
            cc = g * 2 + b
            buf, sem = bufs[b], sems[b]
            cbase = pl.multiple_of(cc * CIDX, 8)
            _drain(table_hbm, idx_v, buf, sem, cbase)
            for r in range(CROWS):
                row = cc * CROWS + r
                a0, a1 = _accum_row(buf, r * H)
                nnz = _count_nnz(idx_v, pl.multiple_of(row * H, 8))
                nnzf = nnz.astype(jnp.float32)
                n0f = jnp.float32(H) - nnzf
                inv = 1.0 / nnzf
                out_v[row, pl.ds(0, 16)] = (a0 - n0f * t0a) * inv
                out_v[row, pl.ds(16, 16)] = (a1 - n0f * t0b) * inv
            nxt = cc + 2

            @pl.when(nxt < NCH)
            def _():
                _issue(table_hbm, idx_v, buf, sem,
                       pl.multiple_of(nxt * CIDX, 8))
        return carry

    lax.fori_loop(0, NCH // 2, outer, 0)
    base = pl.multiple_of(wid * BPW, 8)
    pltpu.sync_copy(out_v, out_hbm.at[pl.ds(base, BPW)])


_doc2vec_sc = functools.partial(
    pl.kernel,
    mesh=plsc.VectorSubcoreMesh(core_axis_name="c", subcore_axis_name="s"),
    compiler_params=pltpu.CompilerParams(use_tc_tiling_on_sc=False),
    out_type=jax.ShapeDtypeStruct((B, D), jnp.float32),
    scratch_types=[
        pltpu.VMEM((IPW + 16,), jnp.int32),
        pltpu.VMEM((CIDX, D), jnp.float32),
        pltpu.VMEM((CIDX, D), jnp.float32),
        pltpu.VMEM((BPW, D), jnp.float32),
        pltpu.VMEM((1, D), jnp.float32),
        pltpu.SemaphoreType.DMA,
        pltpu.SemaphoreType.DMA,
    ],
)(_body)


@jax.jit
def kernel(word_indices, table):
    idx = word_indices.reshape(NW, IPW)
    return _doc2vec_sc(idx, table)
